# baseline (speedup 1.0000x reference)
import functools

import jax
import jax.numpy as jnp
from jax import lax
from jax.experimental import pallas as pl
from jax.experimental.pallas import tpu as pltpu
from jax.experimental.pallas import tpu_sc as plsc

D = 64
GROUPS = D // 16
CHUNK = 16
NW = 32
PER_WORKER = 256
NCHUNKS = PER_WORKER // CHUNK


def _sc_embed_sum(subj_cube, obj_cube, noun_matrix, vs, ns, vo, no):
    mesh = plsc.VectorSubcoreMesh(core_axis_name="c", subcore_axis_name="s")

    @functools.partial(
        pl.kernel,
        out_type=jax.ShapeDtypeStruct((NW, D), jnp.float32),
        mesh=mesh,
        compiler_params=pltpu.CompilerParams(
            needs_layout_passes=False, use_tc_tiling_on_sc=True),
        scratch_types=[
            pltpu.VMEM((PER_WORKER,), jnp.int32),
            pltpu.VMEM((PER_WORKER,), jnp.int32),
            pltpu.VMEM((CHUNK, D * D), jnp.float32),
            pltpu.VMEM((CHUNK, 128), jnp.float32),
            pltpu.VMEM((D,), jnp.float32),
            pltpu.VMEM((2 * 16 * 17,), jnp.float32),
            pltpu.SemaphoreType.DMA,
            pltpu.SemaphoreType.DMA,
        ],
    )
    def k(subj_hbm, obj_hbm, noun_hbm, vs_hbm, ns_hbm, vo_hbm, no_hbm,
          out_hbm, vidx_v, nidx_v, rows_v, vecs_v, acc_v, tile_v,
          rsem, vsem):
        nc = 2
        wid = lax.axis_index("s") * nc + lax.axis_index("c")
        iota = lax.iota(jnp.int32, 16)
        iota17 = iota * 17
        zero = jnp.zeros((16,), jnp.float32)

        tot = (zero, zero, zero, zero)
        for cube_hbm, vidx_hbm, nidx_hbm in (
            (subj_hbm, vs_hbm, ns_hbm),
            (obj_hbm, vo_hbm, no_hbm),
        ):
            base = wid * PER_WORKER
            pltpu.sync_copy(vidx_hbm.at[pl.ds(base, PER_WORKER)], vidx_v)
            pltpu.sync_copy(nidx_hbm.at[pl.ds(base, PER_WORKER)], nidx_v)

            def chunk_body(c, tot, cube_hbm=cube_hbm):
                off = c * CHUNK
                hr = pltpu.async_copy(
                    cube_hbm.at[vidx_v.at[pl.ds(off, CHUNK)]], rows_v, rsem)
                hv = pltpu.async_copy(
                    noun_hbm.at[nidx_v.at[pl.ds(off, CHUNK)]], vecs_v, vsem)
                hr.wait()
                hv.wait()

                def item_body(b, ts):
                    bvec = jnp.full((16,), b, jnp.int32)
                    tb = (b & 1) * 272
                    vq = [plsc.load_gather(vecs_v, [bvec, q * 16 + iota])
                          for q in range(4)]

                    def compute_p(g):
                        P = [zero] * 16
                        for q in range(4):
                            for l in range(16):
                                row = plsc.load_gather(
                                    rows_v,
                                    [bvec, iota + ((16 * g + l) * D + 16 * q)])
                                P[l] = P[l] + row * vq[q]
                        return P

                    outs = []
                    for g in range(GROUPS):
                        P = compute_p(g)
                        for l in range(16):
                            plsc.store_scatter(
                                tile_v, [(iota + 17 * l) + tb], P[l])
                        x = zero
                        for cc in range(16):
                            x = x + plsc.load_gather(
                                tile_v, [(iota17 + cc) + tb])
                        e = jnp.exp(-2.0 * jnp.abs(x))
                        outs.append(
                            ts[g] + jnp.sign(x) * (1.0 - e) / (1.0 + e))
                    return tuple(outs)

                return plsc.parallel_loop(
                    0, CHUNK, unroll=2, carry=tot)(item_body)

            tot = lax.fori_loop(0, NCHUNKS, chunk_body, tot)

        for g in range(GROUPS):
            acc_v[pl.ds(g * 16, 16)] = tot[g]
        pltpu.sync_copy(acc_v, out_hbm.at[wid])

    return k(subj_cube, obj_cube, noun_matrix, vs, ns, vo, no)


def kernel(words, verb_subj, verb_obj, verb_trans, noun_matrix,
           subj_verb_cube, obj_verb_cube):
    i32 = jnp.int32
    vs = jnp.concatenate([verb_subj[:, 0], verb_trans[:, 0]]).astype(i32)
    ns = jnp.concatenate([verb_subj[:, 1], verb_trans[:, 1]]).astype(i32)
    vo = jnp.concatenate([verb_obj[:, 0], verb_trans[:, 0]]).astype(i32)
    no = jnp.concatenate([verb_obj[:, 1], verb_trans[:, 2]]).astype(i32)
    noun_pad = jnp.pad(noun_matrix, ((0, 0), (0, 64)))
    partial = _sc_embed_sum(subj_verb_cube, obj_verb_cube, noun_pad,
                            vs, ns, vo, no)
    return partial.sum(axis=0)


# speedup vs baseline: 2.1371x; 2.1371x over previous
import functools

import jax
import jax.numpy as jnp
from jax import lax
from jax.experimental import pallas as pl
from jax.experimental.pallas import tpu as pltpu
from jax.experimental.pallas import tpu_sc as plsc

D = 64
GROUPS = D // 16
CHUNK = 16
NW = 32
PER_WORKER = 256
NCHUNKS = PER_WORKER // CHUNK


def _sc_embed_sum(subj_cube, obj_cube, noun_matrix, vs, ns, vo, no):
    mesh = plsc.VectorSubcoreMesh(core_axis_name="c", subcore_axis_name="s")

    @functools.partial(
        pl.kernel,
        out_type=jax.ShapeDtypeStruct((NW, D), jnp.float32),
        mesh=mesh,
        compiler_params=pltpu.CompilerParams(
            needs_layout_passes=False, use_tc_tiling_on_sc=True),
        scratch_types=[
            pltpu.VMEM((PER_WORKER,), jnp.int32),
            pltpu.VMEM((PER_WORKER,), jnp.int32),
            pltpu.VMEM((CHUNK, D * D), jnp.float32),
            pltpu.VMEM((CHUNK, 128), jnp.float32),
            pltpu.VMEM((D,), jnp.float32),
            pltpu.VMEM((2 * 16 * 17,), jnp.float32),
            pltpu.SemaphoreType.DMA,
            pltpu.SemaphoreType.DMA,
        ],
    )
    def k(subj_hbm, obj_hbm, noun_hbm, vs_hbm, ns_hbm, vo_hbm, no_hbm,
          out_hbm, vidx_v, nidx_v, rows_v, vecs_v, acc_v, tile_v,
          rsem, vsem):
        nc = 2
        wid = lax.axis_index("s") * nc + lax.axis_index("c")
        iota = lax.iota(jnp.int32, 16)
        iota17 = iota * 17
        zero = jnp.zeros((16,), jnp.float32)

        tot = (zero, zero, zero, zero)
        for cube_hbm, vidx_hbm, nidx_hbm in (
            (subj_hbm, vs_hbm, ns_hbm),
            (obj_hbm, vo_hbm, no_hbm),
        ):
            base = wid * PER_WORKER
            pltpu.sync_copy(vidx_hbm.at[pl.ds(base, PER_WORKER)], vidx_v)
            pltpu.sync_copy(nidx_hbm.at[pl.ds(base, PER_WORKER)], nidx_v)

            def chunk_body(c, tot, cube_hbm=cube_hbm):
                off = c * CHUNK
                hr = pltpu.async_copy(
                    cube_hbm.at[vidx_v.at[pl.ds(off, CHUNK)]], rows_v, rsem)
                hv = pltpu.async_copy(
                    noun_hbm.at[nidx_v.at[pl.ds(off, CHUNK)]], vecs_v, vsem)
                hr.wait()
                hv.wait()

                def item_body(b, ts):
                    bvec = jnp.full((16,), b, jnp.int32)
                    vq = [plsc.load_gather(vecs_v, [bvec, q * 16 + iota])
                          for q in range(4)]

                    def compute_p(g):
                        P = [zero] * 16
                        for q in range(4):
                            for l in range(16):
                                row = plsc.load_gather(
                                    rows_v,
                                    [bvec, iota + ((16 * g + l) * D + 16 * q)])
                                P[l] = P[l] + row * vq[q]
                        return P

                    outs = []
                    for g in range(GROUPS):
                        P = compute_p(g)
                        for l in range(16):
                            plsc.store_scatter(tile_v, [iota + 17 * l], P[l])
                        x = zero
                        for cc in range(16):
                            x = x + plsc.load_gather(tile_v, [iota17 + cc])
                        e = jnp.exp(-2.0 * jnp.abs(x))
                        outs.append(
                            ts[g] + jnp.sign(x) * (1.0 - e) / (1.0 + e))
                    return tuple(outs)

                return lax.fori_loop(0, CHUNK, item_body, tot)

            tot = lax.fori_loop(0, NCHUNKS, chunk_body, tot)

        for g in range(GROUPS):
            acc_v[pl.ds(g * 16, 16)] = tot[g]
        pltpu.sync_copy(acc_v, out_hbm.at[wid])

    return k(subj_cube, obj_cube, noun_matrix, vs, ns, vo, no)


def kernel(words, verb_subj, verb_obj, verb_trans, noun_matrix,
           subj_verb_cube, obj_verb_cube):
    i32 = jnp.int32
    vs = jnp.concatenate([verb_subj[:, 0], verb_trans[:, 0]]).astype(i32)
    ns = jnp.concatenate([verb_subj[:, 1], verb_trans[:, 1]]).astype(i32)
    vo = jnp.concatenate([verb_obj[:, 0], verb_trans[:, 0]]).astype(i32)
    no = jnp.concatenate([verb_obj[:, 1], verb_trans[:, 2]]).astype(i32)
    noun_pad = jnp.pad(noun_matrix, ((0, 0), (0, 64)))
    partial = _sc_embed_sum(subj_verb_cube, obj_verb_cube, noun_pad,
                            vs, ns, vo, no)
    return partial.sum(axis=0)
